# baseline (speedup 1.0000x reference)
import functools

import jax
import jax.numpy as jnp
from jax import lax
from jax.experimental import pallas as pl
from jax.experimental.pallas import tpu as pltpu
from jax.experimental.pallas import tpu_sc as plsc

N_VOCAB = 49408
N_EMBD = 768
N_TOKENS = 77
BATCH = 1024

NC = 2
NS = 16
NW = NC * NS
ROWS = BATCH * N_TOKENS
ROWS_W = ROWS // NW
C = 16
J = ROWS_W // C
NBUF = 4
J_MAIN = (J // NBUF) * NBUF
LANES = 16
G = N_EMBD // LANES


def _embed_body(idx_hbm, table_hbm, pos_hbm, out_hbm,
                idx_v, pos_v, buf0, buf1, buf2, buf3,
                gs0, gs1, gs2, gs3, ss0, ss1, ss2, ss3):
    bufs = (buf0, buf1, buf2, buf3)
    gsems = (gs0, gs1, gs2, gs3)
    ssems = (ss0, ss1, ss2, ss3)

    cid = lax.axis_index("c")
    sid = lax.axis_index("s")
    wid = sid * NC + cid
    row0 = wid * ROWS_W

    pltpu.sync_copy(idx_hbm.at[wid], idx_v)
    pltpu.sync_copy(pos_hbm, pos_v)

    def issue_gather(j, b):
        pltpu.async_copy(table_hbm.at[idx_v.at[j]], bufs[b], gsems[b])

    def wait_gather(b):
        pltpu.make_async_copy(table_hbm.at[pl.ds(0, C)], bufs[b], gsems[b]).wait()

    def issue_store(j, b):
        pltpu.async_copy(bufs[b], out_hbm.at[pl.ds(row0 + j * C, C)], ssems[b])

    def wait_store(b):
        pltpu.make_async_copy(bufs[b], out_hbm.at[pl.ds(0, C)], ssems[b]).wait()

    issue_gather(0, 0)
    issue_gather(1, 1)

    def add_pos(j, b):
        def row_body(i, _):
            p = lax.rem(j * C + i, N_TOKENS)
            for g in range(G):
                sl = pl.ds(g * LANES, LANES)
                plsc.addupdate(bufs[b].at[i, sl], pos_v[p, sl])
            return 0
        lax.fori_loop(0, C, row_body, 0, unroll=False)

    def outer(jo, _):
        for b in range(NBUF):
            j = jo * NBUF + b
            wait_gather(b)
            add_pos(j, b)
            issue_store(j, b)
            k = j + 2
            bk = (b + 2) % NBUF
            @pl.when(j >= 2)
            def _():
                wait_store(bk)

            issue_gather(k, bk)
        return 0

    lax.fori_loop(0, J_MAIN // NBUF, outer, 0, unroll=False)

    for j in range(J_MAIN, J):
        b = j % NBUF
        wait_gather(b)
        add_pos(jnp.int32(j), b)
        issue_store(j, b)

    for j in range(J - NBUF, J):
        wait_store(j % NBUF)


@functools.partial(
    pl.kernel,
    out_type=jax.ShapeDtypeStruct((ROWS, N_EMBD), jnp.float32),
    mesh=plsc.VectorSubcoreMesh(core_axis_name="c", subcore_axis_name="s"),
    scratch_types=[
        pltpu.VMEM((J, C), jnp.int32),
        pltpu.VMEM((N_TOKENS, N_EMBD), jnp.float32),
        pltpu.VMEM((C, N_EMBD), jnp.float32),
        pltpu.VMEM((C, N_EMBD), jnp.float32),
        pltpu.VMEM((C, N_EMBD), jnp.float32),
        pltpu.VMEM((C, N_EMBD), jnp.float32),
        pltpu.SemaphoreType.DMA,
        pltpu.SemaphoreType.DMA,
        pltpu.SemaphoreType.DMA,
        pltpu.SemaphoreType.DMA,
        pltpu.SemaphoreType.DMA,
        pltpu.SemaphoreType.DMA,
        pltpu.SemaphoreType.DMA,
        pltpu.SemaphoreType.DMA,
    ],
)
def _embed_kernel(idx_hbm, table_hbm, pos_hbm, out_hbm, *scratch):
    _embed_body(idx_hbm, table_hbm, pos_hbm, out_hbm, *scratch)


def kernel(tokens, token_embedding, position_embedding):
    idx = jnp.asarray(tokens, jnp.int32).reshape(NW, J, C)
    out = _embed_kernel(idx, token_embedding, position_embedding)
    return out.reshape(BATCH, N_TOKENS, N_EMBD)
